# baseline (speedup 1.0000x reference)
import math

import jax
import jax.numpy as jnp
from jax import lax
from jax.experimental import pallas as pl
from jax.experimental.pallas import tpu as pltpu

ROWS = 128
COLS = 100000
TEMP = 5.0
BC = 4096
NCB = math.ceil(COLS / BC)

_KS0 = 0
_KS1 = 42
_KS2 = 42 ^ 0x1BD11BDA
_ROT_A = (13, 15, 26, 6)
_ROT_B = (17, 29, 16, 24)


def _rounds(x0, x1, rots):
    for r in rots:
        x0 = x0 + x1
        x1 = (x1 << r) | lax.shift_right_logical(x1, 32 - r)
        x1 = x1 ^ x0
    return x0, x1


def _threefry_bits(e):
    x0 = jnp.zeros_like(e) + _KS0
    x1 = e + _KS1
    x0, x1 = _rounds(x0, x1, _ROT_A)
    x0, x1 = x0 + _KS1, x1 + (_KS2 + 1)
    x0, x1 = _rounds(x0, x1, _ROT_B)
    x0, x1 = x0 + _KS2, x1 + (_KS0 + 2)
    x0, x1 = _rounds(x0, x1, _ROT_A)
    x0, x1 = x0 + _KS0, x1 + (_KS1 + 3)
    x0, x1 = _rounds(x0, x1, _ROT_B)
    x0, x1 = x0 + _KS1, x1 + (_KS2 + 4)
    x0, x1 = _rounds(x0, x1, _ROT_A)
    x0, x1 = x0 + _KS2, x1 + (_KS0 + 5)
    return x0 ^ x1


def _argmax_zero_kernel(x_ref, zero_ref, idx_ref, val_ref):
    cb = pl.program_id(0)
    jj = lax.broadcasted_iota(jnp.int32, (ROWS, BC), 1) + cb * BC
    ii = lax.broadcasted_iota(jnp.int32, (ROWS, BC), 0)
    e = ii * COLS + jj
    bits = _threefry_bits(e)
    mant = lax.shift_right_logical(bits, 9) | 0x3F800000
    u = lax.bitcast_convert_type(mant, jnp.float32) - 1.0
    g = -jnp.log(-jnp.log(u + 1e-8) + 1e-8)
    w = (x_ref[...] + g) / TEMP
    w = jnp.where(jj < COLS, w, -jnp.inf)

    zero_ref[...] = jnp.zeros((ROWS, BC), jnp.float32)

    m = jnp.max(w, axis=1, keepdims=True)
    idxb = jnp.min(
        jnp.where(w == m, jj, jnp.int32(2**31 - 1)), axis=1, keepdims=True
    )

    @pl.when(cb == 0)
    def _():
        val_ref[...] = m
        idx_ref[...] = idxb

    @pl.when(cb > 0)
    def _():
        better = m > val_ref[...]
        val_ref[...] = jnp.where(better, m, val_ref[...])
        idx_ref[...] = jnp.where(better, idxb, idx_ref[...])


_NG = ROWS // 8
_LAST_TILE = (COLS // 128) * 128


def _scatter_kernel(idx_sref, idxg_ref, zero_ref, out_ref, patch_ref, sem):
    del zero_ref
    idxg = idxg_ref[...]
    base = (idxg // 128) * 128
    ll = lax.broadcasted_iota(jnp.int32, (_NG, 8, 8, 128), 3)
    tgt = base[:, :, None, None] + ll
    want = idxg[:, None, :, None]
    patch_ref[...] = (tgt == want).astype(jnp.float32)

    def _copy(i):
        g = i // 8
        d = i - g * 8
        c = idx_sref[i, 0]
        b = pl.multiple_of((c // 128) * 128, 128)
        row0 = pl.multiple_of(g * 8, 8)
        return pltpu.make_async_copy(
            patch_ref.at[g, d],
            out_ref.at[pl.ds(row0, 8), pl.ds(b, 128)],
            sem,
        )

    def _start(i, carry):
        _copy(i).start()
        return carry

    def _drain(i, carry):
        _copy(i).wait()
        return carry

    lax.fori_loop(0, ROWS, _start, 0)
    lax.fori_loop(0, ROWS, _drain, 0)


@jax.jit
def kernel(logits):
    zeroed, idx = pl.pallas_call(
        _argmax_zero_kernel,
        grid=(NCB,),
        in_specs=[pl.BlockSpec((ROWS, BC), lambda cb: (0, cb))],
        out_specs=[
            pl.BlockSpec((ROWS, BC), lambda cb: (0, cb)),
            pl.BlockSpec((ROWS, 1), lambda cb: (0, 0)),
        ],
        out_shape=[
            jax.ShapeDtypeStruct((ROWS, COLS), jnp.float32),
            jax.ShapeDtypeStruct((ROWS, 1), jnp.int32),
        ],
        scratch_shapes=[pltpu.VMEM((ROWS, 1), jnp.float32)],
    )(logits)
    out = pl.pallas_call(
        _scatter_kernel,
        in_specs=[
            pl.BlockSpec(memory_space=pltpu.SMEM),
            pl.BlockSpec(memory_space=pltpu.VMEM),
            pl.BlockSpec(memory_space=pl.ANY),
        ],
        out_specs=pl.BlockSpec(memory_space=pl.ANY),
        out_shape=jax.ShapeDtypeStruct((ROWS, COLS), jnp.float32),
        scratch_shapes=[
            pltpu.VMEM((_NG, 8, 8, 128), jnp.float32),
            pltpu.SemaphoreType.DMA,
        ],
        input_output_aliases={2: 0},
    )(idx, idx.reshape(_NG, 8), zeroed)
    return out


# speedup vs baseline: 1.0635x; 1.0635x over previous
import math

import jax
import jax.numpy as jnp
from jax import lax
from jax.experimental import pallas as pl
from jax.experimental.pallas import tpu as pltpu

ROWS = 128
COLS = 100000
TEMP = 5.0
BC = 1024
NCB = math.ceil(COLS / BC)

_KS0 = 0
_KS1 = 42
_KS2 = 42 ^ 0x1BD11BDA
_ROT_A = (13, 15, 26, 6)
_ROT_B = (17, 29, 16, 24)


def _rounds(x0, x1, rots):
    for r in rots:
        x0 = x0 + x1
        x1 = (x1 << r) | lax.shift_right_logical(x1, 32 - r)
        x1 = x1 ^ x0
    return x0, x1


def _threefry_bits(e):
    x0 = jnp.zeros_like(e) + _KS0
    x1 = e + _KS1
    x0, x1 = _rounds(x0, x1, _ROT_A)
    x0, x1 = x0 + _KS1, x1 + (_KS2 + 1)
    x0, x1 = _rounds(x0, x1, _ROT_B)
    x0, x1 = x0 + _KS2, x1 + (_KS0 + 2)
    x0, x1 = _rounds(x0, x1, _ROT_A)
    x0, x1 = x0 + _KS0, x1 + (_KS1 + 3)
    x0, x1 = _rounds(x0, x1, _ROT_B)
    x0, x1 = x0 + _KS1, x1 + (_KS2 + 4)
    x0, x1 = _rounds(x0, x1, _ROT_A)
    x0, x1 = x0 + _KS2, x1 + (_KS0 + 5)
    return x0 ^ x1


def _argmax_zero_kernel(x_ref, zero_ref, idx_ref, val_ref):
    cb = pl.program_id(0)
    jj = lax.broadcasted_iota(jnp.int32, (ROWS, BC), 1) + cb * BC
    ii = lax.broadcasted_iota(jnp.int32, (ROWS, BC), 0)
    e = ii * COLS + jj
    bits = _threefry_bits(e)
    mant = lax.shift_right_logical(bits, 9) | 0x3F800000
    u = lax.bitcast_convert_type(mant, jnp.float32) - 1.0
    g = -jnp.log(-jnp.log(u + 1e-8) + 1e-8)
    w = (x_ref[...] + g) / TEMP
    w = jnp.where(jj < COLS, w, -jnp.inf)

    zero_ref[...] = jnp.zeros((ROWS, BC), jnp.float32)

    m = jnp.max(w, axis=1, keepdims=True)
    idxb = jnp.min(
        jnp.where(w == m, jj, jnp.int32(2**31 - 1)), axis=1, keepdims=True
    )

    @pl.when(cb == 0)
    def _():
        val_ref[...] = m
        idx_ref[...] = idxb

    @pl.when(cb > 0)
    def _():
        better = m > val_ref[...]
        val_ref[...] = jnp.where(better, m, val_ref[...])
        idx_ref[...] = jnp.where(better, idxb, idx_ref[...])


_NG = ROWS // 8
_LAST_TILE = (COLS // 128) * 128


def _scatter_kernel(idx_sref, idxg_ref, zero_ref, out_ref, patch_ref, sem):
    del zero_ref
    idxg = idxg_ref[...]
    base = (idxg // 128) * 128
    ll = lax.broadcasted_iota(jnp.int32, (_NG, 8, 8, 128), 3)
    tgt = base[:, :, None, None] + ll
    want = idxg[:, None, :, None]
    patch_ref[...] = (tgt == want).astype(jnp.float32)

    def _copy(i):
        g = i // 8
        d = i - g * 8
        c = idx_sref[i, 0]
        b = pl.multiple_of((c // 128) * 128, 128)
        row0 = pl.multiple_of(g * 8, 8)
        return pltpu.make_async_copy(
            patch_ref.at[g, d],
            out_ref.at[pl.ds(row0, 8), pl.ds(b, 128)],
            sem,
        )

    def _start(i, carry):
        _copy(i).start()
        return carry

    def _drain(i, carry):
        _copy(i).wait()
        return carry

    lax.fori_loop(0, ROWS, _start, 0)
    lax.fori_loop(0, ROWS, _drain, 0)


@jax.jit
def kernel(logits):
    zeroed, idx = pl.pallas_call(
        _argmax_zero_kernel,
        grid=(NCB,),
        in_specs=[pl.BlockSpec((ROWS, BC), lambda cb: (0, cb))],
        out_specs=[
            pl.BlockSpec((ROWS, BC), lambda cb: (0, cb)),
            pl.BlockSpec((ROWS, 1), lambda cb: (0, 0)),
        ],
        out_shape=[
            jax.ShapeDtypeStruct((ROWS, COLS), jnp.float32),
            jax.ShapeDtypeStruct((ROWS, 1), jnp.int32),
        ],
        scratch_shapes=[pltpu.VMEM((ROWS, 1), jnp.float32)],
    )(logits)
    out = pl.pallas_call(
        _scatter_kernel,
        in_specs=[
            pl.BlockSpec(memory_space=pltpu.SMEM),
            pl.BlockSpec(memory_space=pltpu.VMEM),
            pl.BlockSpec(memory_space=pl.ANY),
        ],
        out_specs=pl.BlockSpec(memory_space=pl.ANY),
        out_shape=jax.ShapeDtypeStruct((ROWS, COLS), jnp.float32),
        scratch_shapes=[
            pltpu.VMEM((_NG, 8, 8, 128), jnp.float32),
            pltpu.SemaphoreType.DMA,
        ],
        input_output_aliases={2: 0},
    )(idx, idx.reshape(_NG, 8), zeroed)
    return out


# speedup vs baseline: 1.0791x; 1.0147x over previous
import math

import jax
import jax.numpy as jnp
from jax import lax
from jax.experimental import pallas as pl
from jax.experimental.pallas import tpu as pltpu

ROWS = 128
COLS = 100000
TEMP = 5.0
BC = 2048
NCB = math.ceil(COLS / BC)

_KS0 = 0
_KS1 = 42
_KS2 = 42 ^ 0x1BD11BDA
_ROT_A = (13, 15, 26, 6)
_ROT_B = (17, 29, 16, 24)


def _rounds(x0, x1, rots):
    for r in rots:
        x0 = x0 + x1
        x1 = (x1 << r) | lax.shift_right_logical(x1, 32 - r)
        x1 = x1 ^ x0
    return x0, x1


def _threefry_bits(e):
    x0 = jnp.zeros_like(e) + _KS0
    x1 = e + _KS1
    x0, x1 = _rounds(x0, x1, _ROT_A)
    x0, x1 = x0 + _KS1, x1 + (_KS2 + 1)
    x0, x1 = _rounds(x0, x1, _ROT_B)
    x0, x1 = x0 + _KS2, x1 + (_KS0 + 2)
    x0, x1 = _rounds(x0, x1, _ROT_A)
    x0, x1 = x0 + _KS0, x1 + (_KS1 + 3)
    x0, x1 = _rounds(x0, x1, _ROT_B)
    x0, x1 = x0 + _KS1, x1 + (_KS2 + 4)
    x0, x1 = _rounds(x0, x1, _ROT_A)
    x0, x1 = x0 + _KS2, x1 + (_KS0 + 5)
    return x0 ^ x1


def _argmax_zero_kernel(x_ref, zero_ref, idx_ref, val_ref):
    cb = pl.program_id(0)
    jj = lax.broadcasted_iota(jnp.int32, (ROWS, BC), 1) + cb * BC
    ii = lax.broadcasted_iota(jnp.int32, (ROWS, BC), 0)
    e = ii * COLS + jj
    bits = _threefry_bits(e)
    mant = lax.shift_right_logical(bits, 9) | 0x3F800000
    u = lax.bitcast_convert_type(mant, jnp.float32) - 1.0
    g = -jnp.log(-jnp.log(u + 1e-8) + 1e-8)
    w = (x_ref[...] + g) / TEMP
    w = jnp.where(jj < COLS, w, -jnp.inf)

    zero_ref[...] = jnp.zeros((ROWS, BC), jnp.float32)

    m = jnp.max(w, axis=1, keepdims=True)
    idxb = jnp.min(
        jnp.where(w == m, jj, jnp.int32(2**31 - 1)), axis=1, keepdims=True
    )

    @pl.when(cb == 0)
    def _():
        val_ref[...] = m
        idx_ref[...] = idxb

    @pl.when(cb > 0)
    def _():
        better = m > val_ref[...]
        val_ref[...] = jnp.where(better, m, val_ref[...])
        idx_ref[...] = jnp.where(better, idxb, idx_ref[...])


_NG = ROWS // 8


def _scatter_kernel(idx_sref, idxg_ref, zero_ref, out_ref, patch_ref, sem):
    del zero_ref
    idxg = idxg_ref[...]
    base = (idxg // 128) * 128
    ll = lax.broadcasted_iota(jnp.int32, (_NG, 8, 8, 128), 3)
    tgt = base[:, :, None, None] + ll
    want = idxg[:, None, :, None]
    patch_ref[...] = (tgt == want).astype(jnp.float32)

    def _copy(i):
        g = i // 8
        d = i - g * 8
        c = idx_sref[i, 0]
        b = pl.multiple_of((c // 128) * 128, 128)
        row0 = pl.multiple_of(g * 8, 8)
        return pltpu.make_async_copy(
            patch_ref.at[g, d],
            out_ref.at[pl.ds(row0, 8), pl.ds(b, 128)],
            sem,
        )

    def _start(i, carry):
        _copy(i).start()
        return carry

    def _drain(i, carry):
        _copy(i).wait()
        return carry

    lax.fori_loop(0, ROWS, _start, 0)
    lax.fori_loop(0, ROWS, _drain, 0)


@jax.jit
def kernel(logits):
    zeroed, idx = pl.pallas_call(
        _argmax_zero_kernel,
        grid=(NCB,),
        in_specs=[pl.BlockSpec((ROWS, BC), lambda cb: (0, cb))],
        out_specs=[
            pl.BlockSpec((ROWS, BC), lambda cb: (0, cb)),
            pl.BlockSpec((ROWS, 1), lambda cb: (0, 0)),
        ],
        out_shape=[
            jax.ShapeDtypeStruct((ROWS, COLS), jnp.float32),
            jax.ShapeDtypeStruct((ROWS, 1), jnp.int32),
        ],
        scratch_shapes=[pltpu.VMEM((ROWS, 1), jnp.float32)],
    )(logits)
    out = pl.pallas_call(
        _scatter_kernel,
        in_specs=[
            pl.BlockSpec(memory_space=pltpu.SMEM),
            pl.BlockSpec(memory_space=pltpu.VMEM),
            pl.BlockSpec(memory_space=pl.ANY),
        ],
        out_specs=pl.BlockSpec(memory_space=pl.ANY),
        out_shape=jax.ShapeDtypeStruct((ROWS, COLS), jnp.float32),
        scratch_shapes=[
            pltpu.VMEM((_NG, 8, 8, 128), jnp.float32),
            pltpu.SemaphoreType.DMA,
        ],
        input_output_aliases={2: 0},
    )(idx, idx.reshape(_NG, 8), zeroed)
    return out
